# baseline (speedup 1.0000x reference)
import functools

import jax
import jax.numpy as jnp
from jax import lax
from jax.experimental import pallas as pl
from jax.experimental.pallas import tpu as pltpu
from jax.experimental.pallas import tpu_sc as plsc

D = 64
C = 256
NBUF = 2
GW = 128
RB = 8192
EPS = 1e-5
PAD = 0
LANES = 16
NWORKERS = 32


def _sc_gather(table, idx, n):
    mesh = plsc.VectorSubcoreMesh(core_axis_name="core", subcore_axis_name="subcore")
    nw = n // NWORKERS
    steps = nw // C

    @functools.partial(
        pl.kernel,
        out_type=jax.ShapeDtypeStruct((n // 2, 4 * D), jnp.float32),
        mesh=mesh,
        compiler_params=pltpu.CompilerParams(
            needs_layout_passes=False, use_tc_tiling_on_sc=False
        ),
        scratch_types=[pltpu.VMEM((C // 2, 4 * D), jnp.float32) for _ in range(NBUF)]
        + [pltpu.VMEM((C, D), jnp.float32) for _ in range(NBUF)]
        + [pltpu.VMEM((C,), jnp.int32) for _ in range(NBUF)]
        + [pltpu.SemaphoreType.DMA for _ in range(2 * NBUF)],
    )
    def run(table_hbm, idx_hbm, out_hbm, *scratch):
        rbuf = scratch[:NBUF]
        gbuf = scratch[NBUF : 2 * NBUF]
        ibuf = scratch[2 * NBUF : 3 * NBUF]
        gsem = scratch[3 * NBUF : 4 * NBUF]
        ssem = scratch[4 * NBUF : 5 * NBUF]

        wid = lax.axis_index("subcore") * 2 + lax.axis_index("core")
        base = wid * nw

        def load_and_gather(k, b):
            pltpu.sync_copy(idx_hbm.at[pl.ds(base + k * C, C)], ibuf[b])
            for j in range(C // GW):
                pltpu.async_copy(
                    table_hbm.at[ibuf[b].at[pl.ds(j * GW, GW)]],
                    gbuf[b].at[pl.ds(j * GW, GW)],
                    gsem[b],
                )

        def wait_gather(b):
            for j in range(C // GW):
                pltpu.make_async_copy(
                    table_hbm.at[ibuf[b].at[pl.ds(j * GW, GW)]],
                    gbuf[b].at[pl.ds(j * GW, GW)],
                    gsem[b],
                ).wait()

        def store(k, b):
            pltpu.async_copy(
                rbuf[b],
                out_hbm.at[pl.ds((base + k * C) // 2, C // 2)],
                ssem[b],
            )

        def wait_store(b):
            pltpu.make_async_copy(
                rbuf[b], out_hbm.at[pl.ds(0, C // 2)], ssem[b]
            ).wait()

        def zero_padding_rows(b):
            @pl.loop(0, C // LANES)
            def _(g):
                ivs = ibuf[b][pl.ds(LANES * g, LANES)]
                haspad = jnp.any(ivs == jnp.int32(PAD))

                @pl.when(haspad)
                def _():
                    mf = jnp.where(ivs != jnp.int32(PAD), 1.0, 0.0)
                    for rr in range(LANES):
                        r = LANES * g + rr
                        bm = lax.broadcast_in_dim(mf[rr], (LANES,), ())
                        for j in range(D // LANES):
                            sl = pl.ds(LANES * j, LANES)
                            gbuf[b][r, sl] = gbuf[b][r, sl] * bm

        def expand(b):
            @pl.loop(0, C // 4)
            def _(p0):
                for u in range(2):
                    p = 2 * p0 + u
                    for j in range(D // LANES):
                        sl = pl.ds(LANES * j, LANES)
                        rbuf[b][p, pl.ds(LANES * j, LANES)] = gbuf[b][2 * p, sl]
                        rbuf[b][p, pl.ds(2 * D + LANES * j, LANES)] = gbuf[b][
                            2 * p + 1, sl
                        ]

        load_and_gather(0, 0)

        @pl.loop(0, steps, step=NBUF)
        def _(k0):
            for b in range(NBUF):
                k = k0 + b
                b_next = (b + 1) % NBUF

                @pl.when(k < steps - 1)
                def _():
                    load_and_gather(k + 1, b_next)

                wait_gather(b)
                zero_padding_rows(b)

                @pl.when(k >= NBUF)
                def _():
                    wait_store(b)

                expand(b)
                store(k, b)

        for t in range(steps - NBUF, steps):
            wait_store(t % NBUF)

    return run(table, idx)


def _tc_layer_norm(emb, gamma, beta, n):
    def body(e_ref, g_ref, b_ref, o_ref):
        e = e_ref[...][:, :D]
        mu = jnp.mean(e, axis=-1, keepdims=True)
        d = e - mu
        var = jnp.mean(d * d, axis=-1, keepdims=True)
        y = d * lax.rsqrt(var + EPS)
        o_ref[...] = y * g_ref[...] + b_ref[...]

    return pl.pallas_call(
        body,
        grid=(n // RB,),
        in_specs=[
            pl.BlockSpec((RB, 2 * D), lambda i: (i, 0)),
            pl.BlockSpec((1, D), lambda i: (0, 0)),
            pl.BlockSpec((1, D), lambda i: (0, 0)),
        ],
        out_specs=pl.BlockSpec((RB, D), lambda i: (i, 0)),
        out_shape=jax.ShapeDtypeStruct((n, D), jnp.float32),
    )(emb, gamma.reshape(1, D), beta.reshape(1, D))


def kernel(x, table, gamma, beta):
    B, L = x.shape
    n = B * L
    emb = _sc_gather(table, x.reshape(n), n)
    out = _tc_layer_norm(emb.reshape(n, 2 * D), gamma, beta, n)
    return out.reshape(B, L, D)


# speedup vs baseline: 1.2480x; 1.2480x over previous
import functools

import jax
import jax.numpy as jnp
from jax import lax
from jax.experimental import pallas as pl
from jax.experimental.pallas import tpu as pltpu
from jax.experimental.pallas import tpu_sc as plsc

D = 64
C = 512
NBUF = 2
GW = 128
RB = 8192
EPS = 1e-5
PAD = 0
LANES = 16
NWORKERS = 32


def _sc_gather(table, idx, n):
    mesh = plsc.VectorSubcoreMesh(core_axis_name="core", subcore_axis_name="subcore")
    nw = n // NWORKERS
    steps = nw // C

    @functools.partial(
        pl.kernel,
        out_type=jax.ShapeDtypeStruct((n, 2 * D), jnp.float32),
        mesh=mesh,
        compiler_params=pltpu.CompilerParams(
            needs_layout_passes=False, use_tc_tiling_on_sc=False
        ),
        scratch_types=[pltpu.VMEM((C, D), jnp.float32) for _ in range(NBUF)]
        + [pltpu.VMEM((C,), jnp.int32) for _ in range(NBUF)]
        + [pltpu.SemaphoreType.DMA for _ in range(2 * NBUF)],
    )
    def run(table_hbm, idx_hbm, out_hbm, *scratch):
        rbuf = scratch[:NBUF]
        ibuf = scratch[NBUF : 2 * NBUF]
        gsem = scratch[2 * NBUF : 3 * NBUF]
        ssem = scratch[3 * NBUF : 4 * NBUF]

        wid = lax.axis_index("subcore") * 2 + lax.axis_index("core")
        base = wid * nw

        def load_and_gather(k, b):
            pltpu.sync_copy(idx_hbm.at[pl.ds(base + k * C, C)], ibuf[b])
            for j in range(C // GW):
                pltpu.async_copy(
                    table_hbm.at[ibuf[b].at[pl.ds(j * GW, GW)]],
                    rbuf[b].at[pl.ds(j * GW, GW)],
                    gsem[b],
                )

        def wait_gather(b):
            for j in range(C // GW):
                pltpu.make_async_copy(
                    table_hbm.at[ibuf[b].at[pl.ds(j * GW, GW)]],
                    rbuf[b].at[pl.ds(j * GW, GW)],
                    gsem[b],
                ).wait()

        def store(k, b):
            pltpu.async_copy(
                rbuf[b],
                out_hbm.at[pl.ds(base + k * C, C), pl.ds(0, D)],
                ssem[b],
            )

        def wait_store(b):
            pltpu.make_async_copy(
                rbuf[b], out_hbm.at[pl.ds(0, C), pl.ds(0, D)], ssem[b]
            ).wait()

        def zero_padding_rows(b):
            @pl.loop(0, C // LANES)
            def _(g):
                ivs = ibuf[b][pl.ds(LANES * g, LANES)]
                haspad = jnp.any(ivs == jnp.int32(PAD))

                @pl.when(haspad)
                def _():
                    mf = jnp.where(ivs != jnp.int32(PAD), 1.0, 0.0)
                    for rr in range(LANES):
                        r = LANES * g + rr
                        bm = lax.broadcast_in_dim(mf[rr], (LANES,), ())
                        for j in range(D // LANES):
                            sl = pl.ds(LANES * j, LANES)
                            rbuf[b][r, sl] = rbuf[b][r, sl] * bm

        load_and_gather(0, 0)

        @pl.loop(0, steps, step=NBUF)
        def _(k0):
            for b in range(NBUF):
                k = k0 + b
                b_next = (b + 1) % NBUF

                @pl.when(k >= NBUF - 1)
                def _():
                    wait_store(b_next)

                @pl.when(k < steps - 1)
                def _():
                    load_and_gather(k + 1, b_next)

                wait_gather(b)
                zero_padding_rows(b)
                store(k, b)

        for t in range(steps - NBUF + 1, steps):
            wait_store(t % NBUF)

    return run(table, idx)


def _tc_layer_norm(emb1d, gamma, beta, n):
    def body(e_ref, g_ref, b_ref, o_ref):
        e = e_ref[...].reshape(RB, 2 * D)[:, :D]
        mu = jnp.mean(e, axis=-1, keepdims=True)
        d = e - mu
        var = jnp.mean(d * d, axis=-1, keepdims=True)
        y = d * lax.rsqrt(var + EPS)
        o_ref[...] = y * g_ref[...] + b_ref[...]

    return pl.pallas_call(
        body,
        grid=(n // RB,),
        in_specs=[
            pl.BlockSpec((RB * 2 * D,), lambda i: (i,)),
            pl.BlockSpec((1, D), lambda i: (0, 0)),
            pl.BlockSpec((1, D), lambda i: (0, 0)),
        ],
        out_specs=pl.BlockSpec((RB, D), lambda i: (i, 0)),
        out_shape=jax.ShapeDtypeStruct((n, D), jnp.float32),
    )(emb1d, gamma.reshape(1, D), beta.reshape(1, D))


def kernel(x, table, gamma, beta):
    B, L = x.shape
    n = B * L
    emb = _sc_gather(table, x.reshape(n), n)
    out = _tc_layer_norm(emb.reshape(n * 2 * D), gamma, beta, n)
    return out.reshape(B, L, D)
